# baseline (speedup 1.0000x reference)
import jax
import jax.numpy as jnp
from jax import lax
from jax.experimental import pallas as pl
from jax.experimental.pallas import tpu as pltpu
from jax.experimental.pallas import tpu_sc as plsc

NC = 2
NS = 16
NW = NC * NS
CHUNK = 128


def _make_segsum(n_pad, ch_a, ch_b, d, with_counts):
    rows_per = n_pad // NS
    ch = max(ch_a, ch_b)
    chh = (ch + 1) // 2
    mesh = plsc.VectorSubcoreMesh(core_axis_name="c", subcore_axis_name="s")
    out_type = [jax.ShapeDtypeStruct((NC, n_pad, d), jnp.float32)]
    if with_counts:
        out_type.append(jax.ShapeDtypeStruct((NC * n_pad,), jnp.float32))
    scratch = [
        pltpu.VMEM_SHARED((n_pad, d), jnp.float32),
        pltpu.VMEM((ch, CHUNK), jnp.int32),
        pltpu.VMEM((ch, CHUNK), jnp.int32),
        pltpu.VMEM((CHUNK, d), jnp.float32),
        pltpu.VMEM((16, d), jnp.float32),
        pltpu.SemaphoreType.DMA,
    ]
    if with_counts:
        scratch += [
            pltpu.VMEM_SHARED((n_pad,), jnp.float32),
            pltpu.VMEM((CHUNK,), jnp.float32),
        ]

    def body(h_hbm, src_hbm, dst_hbm, *rest):
        if with_counts:
            (s_hbm, c_hbm, s_sh, src_v, dst_v, rows_v, zbuf, gsem,
             c_sh, ones_v) = rest
        else:
            s_hbm, s_sh, src_v, dst_v, rows_v, zbuf, gsem = rest
        cid = lax.axis_index("c")
        sid = lax.axis_index("s")
        wid = cid * NS + sid
        row0 = sid * rows_per
        mych = jnp.where(cid == 0, ch_a, ch_b)

        with jax.named_scope("zero"):
            zero16 = jnp.zeros((16,), jnp.float32)

            def zsloop(r, carry):
                for k in range(d // 16):
                    zbuf[r, pl.ds(k * 16, 16)] = zero16
                return carry
            lax.fori_loop(0, 16, zsloop, 0)
            for i in range(rows_per // 16):
                pltpu.sync_copy(zbuf, s_sh.at[pl.ds(row0 + i * 16, 16)])
            if with_counts:
                one16 = jnp.ones((16,), jnp.float32)
                for k in range(CHUNK // 16):
                    ones_v[pl.ds(k * 16, 16)] = one16
                for k in range(rows_per // d):
                    pltpu.sync_copy(zbuf.at[0],
                                    c_sh.at[pl.ds(row0 + k * d, d)])
        plsc.subcore_barrier()

        with jax.named_scope("edges"):
            pltpu.sync_copy(src_hbm.at[wid], src_v)
            pltpu.sync_copy(dst_hbm.at[wid], dst_v)
            pltpu.async_copy(h_hbm.at[src_v.at[0]], rows_v, gsem)

            def eloop(j, carry):
                pltpu.make_async_copy(h_hbm.at[src_v.at[j]], rows_v, gsem
                                      ).wait()
                pltpu.sync_copy(rows_v, s_sh.at[dst_v.at[j]], add=True)

                @pl.when(j + 1 < mych)
                def _():
                    pltpu.async_copy(h_hbm.at[src_v.at[j + 1]], rows_v, gsem)
                if with_counts:
                    pltpu.sync_copy(ones_v, c_sh.at[dst_v.at[j]], add=True)
                return carry
            lax.fori_loop(0, mych, eloop, 0)
        plsc.subcore_barrier()

        with jax.named_scope("writeout"):
            pltpu.sync_copy(s_sh.at[pl.ds(row0, rows_per)],
                            s_hbm.at[cid, pl.ds(row0, rows_per)])
            if with_counts:
                pltpu.sync_copy(c_sh.at[pl.ds(row0, rows_per)],
                                c_hbm.at[pl.ds(cid * n_pad + row0,
                                               rows_per)])

    return pl.kernel(body, out_type=tuple(out_type), mesh=mesh,
                     scratch_types=scratch)


def _dot_t(a, w):
    return lax.dot_general(a, w, (((1,), (1,)), ((), ())),
                           preferred_element_type=jnp.float32,
                           precision=lax.Precision.HIGHEST)


def _sage_dense(s_ref, c_ref, h_ref, wl_ref, bl_ref, wr_ref, sc_ref, sh_ref,
                o_ref):
    c2 = c_ref[...]
    s2 = s_ref[...]
    cnt = c2[0] + c2[1]
    inv = 1.0 / jnp.maximum(cnt, 1.0)
    mean = (s2[0] + s2[1]) * inv
    z = _dot_t(mean, wl_ref[...]) + _dot_t(h_ref[...], wr_ref[...])
    z = z + bl_ref[...]
    o_ref[...] = jnp.maximum(z * sc_ref[...] + sh_ref[...], 0.0)


def _sage_dense_mlp(s_ref, c_ref, h_ref, wl_ref, bl_ref, wr_ref, sc_ref,
                    sh_ref, w1_ref, b1_ref, w2_ref, b2_ref, w3_ref, b3_ref,
                    o_ref):
    c2 = c_ref[...]
    s2 = s_ref[...]
    cnt = c2[0] + c2[1]
    inv = 1.0 / jnp.maximum(cnt, 1.0)
    mean = (s2[0] + s2[1]) * inv
    z = _dot_t(mean, wl_ref[...]) + _dot_t(h_ref[...], wr_ref[...])
    z = z + bl_ref[...]
    h3 = jnp.maximum(z * sc_ref[...] + sh_ref[...], 0.0)
    h4 = jnp.maximum(_dot_t(h3, w1_ref[...]) + b1_ref[...], 0.0)
    h5 = jnp.maximum(_dot_t(h4, w2_ref[...]) + b2_ref[...], 0.0)
    z3 = jax.nn.sigmoid(_dot_t(h5, w3_ref[...]) + b3_ref[...])
    o_ref[...] = z3[:, 0:1]


def kernel(x, edge_index, Wl_0, bl_0, Wr_0, gamma_0, beta_0,
           Wl_1, bl_1, Wr_1, gamma_1, beta_1,
           Wl_2, bl_2, Wr_2, gamma_2, beta_2,
           pW1, pb1, pW2, pb2, pW3, pb3):
    n, d = x.shape
    e = edge_index.shape[1]
    h = pW1.shape[0]
    n_pad = -(-(n + 1) // 2048) * 2048
    ch_tot = -(-e // (NS * CHUNK))
    ch_a = max(2, int(round(ch_tot * 0.40)))
    ch_b = ch_tot - ch_a
    ch = -(-max(ch_a, ch_b) // 2) * 2
    e_pad = NS * ch_tot * CHUNK

    src = edge_index[0]
    dst = edge_index[1]
    pad = e_pad - e
    srcq = jnp.concatenate([src, jnp.zeros((pad,), src.dtype)])
    dstq = jnp.concatenate([dst, jnp.full((pad,), n, dst.dtype)])
    ea = NS * ch_a * CHUNK

    def slab(q, fill):
        qa = q[:ea].reshape(NS, ch_a, CHUNK)
        qb = q[ea:].reshape(NS, ch_b, CHUNK)
        full = jnp.full((NS, ch, CHUNK), fill, q.dtype)
        qa = full.at[:, :ch_a].set(qa)
        qb = full.at[:, :ch_b].set(qb)
        return jnp.concatenate([qa, qb])

    srcp = slab(srcq, 0)
    dstp = slab(dstq, n)

    h0 = jnp.concatenate([x, jnp.zeros((n_pad - n, d), x.dtype)])

    bn = 1.0 / jnp.sqrt(jnp.float32(1.0 + 1e-5))
    params = [(Wl_0, bl_0, Wr_0, gamma_0, beta_0),
              (Wl_1, bl_1, Wr_1, gamma_1, beta_1),
              (Wl_2, bl_2, Wr_2, gamma_2, beta_2)]

    segsum_c = _make_segsum(n_pad, ch_a, ch_b, d, True)
    segsum = _make_segsum(n_pad, ch_a, ch_b, d, False)

    s_shape = jax.ShapeDtypeStruct((n_pad, h), jnp.float32)

    hh = h0
    cnt = None
    for i, (wl, bl, wr, g, b) in enumerate(params):
        if i == 0:
            s, c = segsum_c(hh, srcp, dstp)
            cnt = c.reshape(NC, n_pad, 1)
        else:
            (s,) = segsum(hh, srcp, dstp)
        scale = (g * bn).reshape(1, h)
        shift = b.reshape(1, h)
        blr = bl.reshape(1, h)
        blk = 2048
        grid = (n_pad // blk,)
        row_spec = lambda r, c_: pl.BlockSpec((r, c_), lambda i: (i, 0))
        full2 = lambda a: pl.BlockSpec(a.shape, lambda i: (0, 0))
        base_specs = [
            pl.BlockSpec((NC, blk, d), lambda i: (0, i, 0)),
            pl.BlockSpec((NC, blk, 1), lambda i: (0, i, 0)),
            row_spec(blk, d),
            full2(wl), full2(blr), full2(wr), full2(scale), full2(shift),
        ]
        if i < 2:
            hh = pl.pallas_call(
                _sage_dense, out_shape=s_shape, grid=grid,
                in_specs=base_specs,
                out_specs=row_spec(blk, h))(
                s, cnt, hh, wl, blr, wr, scale, shift)
        else:
            pb1r = pb1.reshape(1, h)
            pb2r = pb2.reshape(1, h // 2)
            pW3p = jnp.concatenate([pW3, jnp.zeros((h - 1, h // 2),
                                                   jnp.float32)])
            pb3r = jnp.pad(pb3.reshape(1, 1), ((0, 0), (0, h - 1)))
            out = pl.pallas_call(
                _sage_dense_mlp,
                out_shape=jax.ShapeDtypeStruct((n_pad, 1), jnp.float32),
                grid=grid,
                in_specs=base_specs + [full2(pW1), full2(pb1r), full2(pW2),
                                       full2(pb2r), full2(pW3p), full2(pb3r)],
                out_specs=row_spec(blk, 1))(
                s, cnt, hh, wl, blr, wr, scale, shift,
                pW1, pb1r, pW2, pb2r, pW3p, pb3r)
    return out[:n]


# speedup vs baseline: 1.0207x; 1.0207x over previous
import jax
import jax.numpy as jnp
from jax import lax
from jax.experimental import pallas as pl
from jax.experimental.pallas import tpu as pltpu
from jax.experimental.pallas import tpu_sc as plsc

NC = 2
NS = 16
NW = NC * NS
CHUNK = 128


def _make_segsum(n_pad, ch_a, ch_b, d, with_counts):
    rows_per = n_pad // NS
    ch = max(ch_a, ch_b)
    chh = (ch + 1) // 2
    mesh = plsc.VectorSubcoreMesh(core_axis_name="c", subcore_axis_name="s")
    out_type = [jax.ShapeDtypeStruct((NC, n_pad, d), jnp.float32)]
    if with_counts:
        out_type.append(jax.ShapeDtypeStruct((NC * n_pad,), jnp.float32))
    scratch = [
        pltpu.VMEM_SHARED((n_pad, d), jnp.float32),
        pltpu.VMEM((ch, CHUNK), jnp.int32),
        pltpu.VMEM((ch, CHUNK), jnp.int32),
        pltpu.VMEM((CHUNK, d), jnp.float32),
        pltpu.VMEM((16, d), jnp.float32),
        pltpu.SemaphoreType.DMA,
    ]
    if with_counts:
        scratch += [
            pltpu.VMEM_SHARED((n_pad,), jnp.float32),
            pltpu.VMEM((CHUNK,), jnp.float32),
        ]

    def body(h_hbm, src_hbm, dst_hbm, *rest):
        if with_counts:
            (s_hbm, c_hbm, s_sh, src_v, dst_v, rows_v, zbuf, gsem,
             c_sh, ones_v) = rest
        else:
            s_hbm, s_sh, src_v, dst_v, rows_v, zbuf, gsem = rest
        cid = lax.axis_index("c")
        sid = lax.axis_index("s")
        wid = cid * NS + sid
        row0 = sid * rows_per
        mych = jnp.where(cid == 0, ch_a, ch_b)

        with jax.named_scope("zero"):
            zero16 = jnp.zeros((16,), jnp.float32)

            def zsloop(r, carry):
                for k in range(d // 16):
                    zbuf[r, pl.ds(k * 16, 16)] = zero16
                return carry
            lax.fori_loop(0, 16, zsloop, 0)
            for i in range(rows_per // 16):
                pltpu.sync_copy(zbuf, s_sh.at[pl.ds(row0 + i * 16, 16)])
            if with_counts:
                one16 = jnp.ones((16,), jnp.float32)
                for k in range(CHUNK // 16):
                    ones_v[pl.ds(k * 16, 16)] = one16
                for k in range(rows_per // d):
                    pltpu.sync_copy(zbuf.at[0],
                                    c_sh.at[pl.ds(row0 + k * d, d)])
        plsc.subcore_barrier()

        with jax.named_scope("edges"):
            pltpu.sync_copy(src_hbm.at[wid], src_v)
            pltpu.sync_copy(dst_hbm.at[wid], dst_v)
            pltpu.async_copy(h_hbm.at[src_v.at[0]], rows_v, gsem)

            def eloop(j, carry):
                pltpu.make_async_copy(h_hbm.at[src_v.at[j]], rows_v, gsem
                                      ).wait()
                pltpu.sync_copy(rows_v, s_sh.at[dst_v.at[j]], add=True)

                @pl.when(j + 1 < mych)
                def _():
                    pltpu.async_copy(h_hbm.at[src_v.at[j + 1]], rows_v, gsem)
                if with_counts:
                    pltpu.sync_copy(ones_v, c_sh.at[dst_v.at[j]], add=True)
                return carry
            lax.fori_loop(0, mych, eloop, 0)
        plsc.subcore_barrier()

        with jax.named_scope("writeout"):
            pltpu.sync_copy(s_sh.at[pl.ds(row0, rows_per)],
                            s_hbm.at[cid, pl.ds(row0, rows_per)])
            if with_counts:
                pltpu.sync_copy(c_sh.at[pl.ds(row0, rows_per)],
                                c_hbm.at[pl.ds(cid * n_pad + row0,
                                               rows_per)])

    return pl.kernel(body, out_type=tuple(out_type), mesh=mesh,
                     scratch_types=scratch)


def _dot_t(a, w):
    return lax.dot_general(a, w, (((1,), (1,)), ((), ())),
                           preferred_element_type=jnp.float32,
                           precision=lax.Precision.HIGHEST)


def _sage_dense(s_ref, c_ref, h_ref, wl_ref, bl_ref, wr_ref, sc_ref, sh_ref,
                o_ref):
    c2 = c_ref[...]
    s2 = s_ref[...]
    cnt = c2[0] + c2[1]
    inv = 1.0 / jnp.maximum(cnt, 1.0)
    mean = (s2[0] + s2[1]) * inv
    z = _dot_t(mean, wl_ref[...]) + _dot_t(h_ref[...], wr_ref[...])
    z = z + bl_ref[...]
    o_ref[...] = jnp.maximum(z * sc_ref[...] + sh_ref[...], 0.0)


def _sage_dense_mlp(s_ref, c_ref, h_ref, wl_ref, bl_ref, wr_ref, sc_ref,
                    sh_ref, w1_ref, b1_ref, w2_ref, b2_ref, w3_ref, b3_ref,
                    o_ref):
    c2 = c_ref[...]
    s2 = s_ref[...]
    cnt = c2[0] + c2[1]
    inv = 1.0 / jnp.maximum(cnt, 1.0)
    mean = (s2[0] + s2[1]) * inv
    z = _dot_t(mean, wl_ref[...]) + _dot_t(h_ref[...], wr_ref[...])
    z = z + bl_ref[...]
    h3 = jnp.maximum(z * sc_ref[...] + sh_ref[...], 0.0)
    h4 = jnp.maximum(_dot_t(h3, w1_ref[...]) + b1_ref[...], 0.0)
    h5 = jnp.maximum(_dot_t(h4, w2_ref[...]) + b2_ref[...], 0.0)
    z3 = jax.nn.sigmoid(_dot_t(h5, w3_ref[...]) + b3_ref[...])
    o_ref[...] = z3[:, 0:1]


def kernel(x, edge_index, Wl_0, bl_0, Wr_0, gamma_0, beta_0,
           Wl_1, bl_1, Wr_1, gamma_1, beta_1,
           Wl_2, bl_2, Wr_2, gamma_2, beta_2,
           pW1, pb1, pW2, pb2, pW3, pb3):
    n, d = x.shape
    e = edge_index.shape[1]
    h = pW1.shape[0]
    n_pad = -(-(n + 1) // 2048) * 2048
    ch_tot = -(-e // (NS * CHUNK))
    ch_a = max(2, int(round(ch_tot * 0.36)))
    ch_b = ch_tot - ch_a
    ch = -(-max(ch_a, ch_b) // 2) * 2
    e_pad = NS * ch_tot * CHUNK

    src = edge_index[0]
    dst = edge_index[1]
    pad = e_pad - e
    srcq = jnp.concatenate([src, jnp.zeros((pad,), src.dtype)])
    dstq = jnp.concatenate([dst, jnp.full((pad,), n, dst.dtype)])
    ea = NS * ch_a * CHUNK

    def slab(q, fill):
        qa = q[:ea].reshape(NS, ch_a, CHUNK)
        qb = q[ea:].reshape(NS, ch_b, CHUNK)
        full = jnp.full((NS, ch, CHUNK), fill, q.dtype)
        qa = full.at[:, :ch_a].set(qa)
        qb = full.at[:, :ch_b].set(qb)
        return jnp.concatenate([qa, qb])

    srcp = slab(srcq, 0)
    dstp = slab(dstq, n)

    h0 = jnp.concatenate([x, jnp.zeros((n_pad - n, d), x.dtype)])

    bn = 1.0 / jnp.sqrt(jnp.float32(1.0 + 1e-5))
    params = [(Wl_0, bl_0, Wr_0, gamma_0, beta_0),
              (Wl_1, bl_1, Wr_1, gamma_1, beta_1),
              (Wl_2, bl_2, Wr_2, gamma_2, beta_2)]

    segsum_c = _make_segsum(n_pad, ch_a, ch_b, d, True)
    segsum = _make_segsum(n_pad, ch_a, ch_b, d, False)

    s_shape = jax.ShapeDtypeStruct((n_pad, h), jnp.float32)

    hh = h0
    cnt = None
    for i, (wl, bl, wr, g, b) in enumerate(params):
        if i == 0:
            s, c = segsum_c(hh, srcp, dstp)
            cnt = c.reshape(NC, n_pad, 1)
        else:
            (s,) = segsum(hh, srcp, dstp)
        scale = (g * bn).reshape(1, h)
        shift = b.reshape(1, h)
        blr = bl.reshape(1, h)
        blk = 2048
        grid = (n_pad // blk,)
        row_spec = lambda r, c_: pl.BlockSpec((r, c_), lambda i: (i, 0))
        full2 = lambda a: pl.BlockSpec(a.shape, lambda i: (0, 0))
        base_specs = [
            pl.BlockSpec((NC, blk, d), lambda i: (0, i, 0)),
            pl.BlockSpec((NC, blk, 1), lambda i: (0, i, 0)),
            row_spec(blk, d),
            full2(wl), full2(blr), full2(wr), full2(scale), full2(shift),
        ]
        if i < 2:
            hh = pl.pallas_call(
                _sage_dense, out_shape=s_shape, grid=grid,
                in_specs=base_specs,
                out_specs=row_spec(blk, h))(
                s, cnt, hh, wl, blr, wr, scale, shift)
        else:
            pb1r = pb1.reshape(1, h)
            pb2r = pb2.reshape(1, h // 2)
            pW3p = jnp.concatenate([pW3, jnp.zeros((h - 1, h // 2),
                                                   jnp.float32)])
            pb3r = jnp.pad(pb3.reshape(1, 1), ((0, 0), (0, h - 1)))
            out = pl.pallas_call(
                _sage_dense_mlp,
                out_shape=jax.ShapeDtypeStruct((n_pad, 1), jnp.float32),
                grid=grid,
                in_specs=base_specs + [full2(pW1), full2(pb1r), full2(pW2),
                                       full2(pb2r), full2(pW3p), full2(pb3r)],
                out_specs=row_spec(blk, 1))(
                s, cnt, hh, wl, blr, wr, scale, shift,
                pW1, pb1r, pW2, pb2r, pW3p, pb3r)
    return out[:n]
